# baseline (speedup 1.0000x reference)
import jax
import jax.numpy as jnp
from jax.experimental import pallas as pl
from jax.experimental.pallas import tpu as pltpu

_TOTAL_COMPONENTS = 128
_TOTAL_NODES = 50000
_TOTAL_EDGES = 800000

_GRID = 2
_FB = 20000
_ELB = 320000


def kernel(node_features, edge_index, node_sizes, edge_sizes):
    num_nodes, d = node_features.shape
    num_edges = edge_index.shape[1]
    num_components = node_sizes.shape[0]
    pad_nodes = _TOTAL_NODES - num_nodes
    pad_edges = _TOTAL_EDGES - num_edges

    def body(nf_ref, ei_ref, pf_ref, pei_ref, zfill, efill, sems):
        i = pl.program_id(0)

        def fill_copies():
            return (
                pltpu.make_async_copy(
                    zfill, pf_ref.at[pl.ds(num_nodes, pad_nodes)],
                    sems.at[2]),
                pltpu.make_async_copy(
                    efill, pei_ref.at[:, pl.ds(num_edges, pad_edges)],
                    sems.at[3]),
            )

        @pl.when(i == 0)
        def _():
            zfill[...] = jnp.zeros_like(zfill)
            efill[...] = jnp.full_like(efill, num_nodes)
            zc, ec = fill_copies()
            zc.start()
            ec.start()

        fc = pltpu.make_async_copy(
            nf_ref, pf_ref.at[pl.ds(i * _FB, _FB)], sems.at[0])
        ecopy = pltpu.make_async_copy(
            ei_ref, pei_ref.at[:, pl.ds(i * _ELB, _ELB)], sems.at[1])
        fc.start()
        ecopy.start()
        fc.wait()
        ecopy.wait()

        @pl.when(i == _GRID - 1)
        def _():
            zc, ec = fill_copies()
            zc.wait()
            ec.wait()

    padded_features, padded_edge_index = pl.pallas_call(
        body,
        grid=(_GRID,),
        out_shape=[
            jax.ShapeDtypeStruct((_TOTAL_NODES, d), node_features.dtype),
            jax.ShapeDtypeStruct((2, _TOTAL_EDGES), edge_index.dtype),
        ],
        in_specs=[
            pl.BlockSpec((_FB, d), lambda i: (i, 0)),
            pl.BlockSpec((2, _ELB), lambda i: (0, i)),
        ],
        out_specs=[
            pl.BlockSpec(memory_space=pl.ANY),
            pl.BlockSpec(memory_space=pl.ANY),
        ],
        scratch_shapes=[
            pltpu.VMEM((pad_nodes, d), node_features.dtype),
            pltpu.VMEM((2, pad_edges), edge_index.dtype),
            pltpu.SemaphoreType.DMA((4,)),
        ],
    )(node_features, edge_index)

    padded_node_sizes = (
        jnp.zeros((_TOTAL_COMPONENTS,), dtype=node_sizes.dtype)
        .at[:num_components].set(node_sizes)
        .at[num_components].set(jnp.asarray(pad_nodes, node_sizes.dtype)))
    padded_edge_sizes = (
        jnp.zeros((_TOTAL_COMPONENTS,), dtype=edge_sizes.dtype)
        .at[:num_components].set(edge_sizes)
        .at[num_components].set(jnp.asarray(pad_edges, edge_sizes.dtype)))
    component_mask = jnp.arange(_TOTAL_COMPONENTS) < num_components

    return (
        padded_features,
        padded_edge_index,
        padded_node_sizes,
        padded_edge_sizes,
        component_mask,
    )
